# baseline (speedup 1.0000x reference)
import functools

import jax
import jax.numpy as jnp
from jax import lax
from jax.experimental import pallas as pl
from jax.experimental.pallas import tpu as pltpu
from jax.experimental.pallas import tpu_sc as plsc

N = 10000
E = 320000
D = 128
C = 40
G = 64

NPX = 10112
NCHUNK = 80
EP = 32 * NCHUNK * 128
NW = 32
RPT = NPX // 16
K0 = 40
K1 = 120

_HIGHEST = jax.lax.Precision.HIGHEST


@functools.lru_cache(maxsize=None)
def _sc_mesh():
    return plsc.VectorSubcoreMesh(core_axis_name="c", subcore_axis_name="s")


def _deg_body(dst_hbm, out_hbm, idx_d, ones_v, zeros_v, acc):
    c = lax.axis_index("c")
    s = lax.axis_index("s")
    wid = s * 2 + c
    pltpu.sync_copy(dst_hbm.at[wid], idx_d)
    for k in range(8):
        ones_v[pl.ds(k * 16, 16)] = jnp.ones((16,), jnp.float32)

    def _zfill(t, carry):
        zeros_v[pl.ds(t * 16, 16)] = jnp.zeros((16,), jnp.float32)
        return carry

    lax.fori_loop(0, 40, _zfill, 0)
    base = jnp.minimum(s * RPT, NPX - 640)
    pltpu.sync_copy(zeros_v, acc.at[pl.ds(base, 640)])
    plsc.subcore_barrier()

    def _scat(j, carry):
        pltpu.sync_copy(ones_v, acc.at[idx_d.at[j]], add=True)
        return carry

    lax.fori_loop(0, NCHUNK, _scat, 0)
    plsc.subcore_barrier()
    pltpu.sync_copy(acc.at[pl.ds(base, 640)], zeros_v)
    pltpu.sync_copy(zeros_v, out_hbm.at[pl.ds(c * NPX + base, 640)])


@functools.lru_cache(maxsize=None)
def _deg_kernel():
    return pl.kernel(
        _deg_body,
        out_type=jax.ShapeDtypeStruct((2 * NPX,), jnp.float32),
        mesh=_sc_mesh(),
        scratch_types=[
            pltpu.VMEM((NCHUNK, 128), jnp.int32),
            pltpu.VMEM((128,), jnp.float32),
            pltpu.VMEM((640,), jnp.float32),
            pltpu.VMEM_SHARED((NPX,), jnp.float32),
        ],
    )


def _scatter_body(p_hbm, sd_hbm, out_hbm, idx, rows, acc, gsem, ssem):
    c = lax.axis_index("c")
    s = lax.axis_index("s")
    wid = s * 2 + c

    def _zrow(r, carry):
        for k in range(8):
            rows[r, pl.ds(k * 16, 16)] = jnp.zeros((16,), jnp.float32)
        return carry

    lax.fori_loop(0, 128, _zrow, 0)

    def _zcopy(k, carry):
        pltpu.sync_copy(rows.at[pl.ds(0, 128)],
                        acc.at[pl.ds(s * RPT + k * 128, 128)])
        return carry

    lax.fori_loop(0, 4, _zcopy, 0)
    pltpu.sync_copy(rows.at[pl.ds(0, 120)],
                    acc.at[pl.ds(s * RPT + 512, 120)])
    plsc.subcore_barrier()

    rows_a = rows.at[pl.ds(0, 128)]
    rows_b = rows.at[pl.ds(128, 128)]
    start2 = pl.multiple_of(s * 320 + c * (2 * K0), 8)
    nph = jnp.where(c == 0, K0 // 8, K1 // 8)

    def _phase(ph, carry):
        off = pl.multiple_of(ph * 16, 16)
        pltpu.sync_copy(sd_hbm.at[pl.ds(start2 + off, 16)], idx)

        def _group(j, carry2):
            @pl.when((j > 0) | (ph > 0))
            def _():
                pltpu.make_async_copy(rows, acc.at[pl.ds(0, 256)],
                                      ssem).wait()

            r = j * 4
            pltpu.async_copy(p_hbm.at[idx.at[r]], rows_a, gsem)
            pltpu.async_copy(p_hbm.at[idx.at[r + 2]], rows_b, gsem)
            pltpu.make_async_copy(p_hbm.at[pl.ds(0, 256)], rows, gsem).wait()
            pltpu.async_copy(rows_a, acc.at[idx.at[r + 1]], ssem, add=True)
            pltpu.async_copy(rows_b, acc.at[idx.at[r + 3]], ssem, add=True)
            return carry2

        lax.fori_loop(0, 4, _group, 0)
        return carry

    lax.fori_loop(0, nph, _phase, 0)
    pltpu.make_async_copy(rows, acc.at[pl.ds(0, 256)], ssem).wait()
    plsc.subcore_barrier()

    def _wb(k, carry):
        pltpu.sync_copy(acc.at[pl.ds(s * RPT + k * 128, 128)],
                        out_hbm.at[c, pl.ds(s * RPT + k * 128, 128)])
        return carry

    lax.fori_loop(0, 4, _wb, 0)
    pltpu.sync_copy(acc.at[pl.ds(s * RPT + 512, 120)],
                    out_hbm.at[c, pl.ds(s * RPT + 512, 120)])


@functools.lru_cache(maxsize=None)
def _scatter_kernel():
    return pl.kernel(
        _scatter_body,
        out_type=jax.ShapeDtypeStruct((2, NPX, D), jnp.float32),
        mesh=_sc_mesh(),
        scratch_types=[
            pltpu.VMEM((16, 128), jnp.int32),
            pltpu.VMEM((256, D), jnp.float32),
            pltpu.VMEM_SHARED((NPX, D), jnp.float32),
            pltpu.SemaphoreType.DMA,
            pltpu.SemaphoreType.DMA,
        ],
    )


def _first_body(d0_ref, d1_ref, x_ref, w_ref, p_ref, dinv_ref):
    i = pl.program_id(0)
    deg = d0_ref[...] + d1_ref[...] + 1.0
    row = lax.broadcasted_iota(jnp.int32, (128, 1), 0) + i * 128
    dinv = jnp.where(row < N, lax.rsqrt(deg), 0.0)
    hw = jnp.dot(x_ref[...], w_ref[...],
                 preferred_element_type=jnp.float32, precision=_HIGHEST)
    p_ref[...] = dinv * hw
    dinv_ref[...] = dinv


def _first_call(d0, d1, x, w):
    return pl.pallas_call(
        _first_body,
        grid=(NPX // 128,),
        in_specs=[
            pl.BlockSpec((128, 1), lambda i: (i, 0)),
            pl.BlockSpec((128, 1), lambda i: (i, 0)),
            pl.BlockSpec((128, D), lambda i: (i, 0)),
            pl.BlockSpec((D, D), lambda i: (0, 0)),
        ],
        out_specs=[
            pl.BlockSpec((128, D), lambda i: (i, 0)),
            pl.BlockSpec((128, 1), lambda i: (i, 0)),
        ],
        out_shape=[
            jax.ShapeDtypeStruct((NPX, D), jnp.float32),
            jax.ShapeDtypeStruct((NPX, 1), jnp.float32),
        ],
    )(d0, d1, x, w)


def _mid_body(s_ref, p_ref, dinv_ref, b_ref, w_ref, o_ref):
    dinv = dinv_ref[...]
    h = jnp.maximum(dinv * (s_ref[0] + s_ref[1] + p_ref[...]) + b_ref[...],
                    0.0)
    o_ref[...] = dinv * jnp.dot(h, w_ref[...],
                                preferred_element_type=jnp.float32,
                                precision=_HIGHEST)


def _mid_call(sv, p, dinv, b, w):
    return pl.pallas_call(
        _mid_body,
        grid=(NPX // 128,),
        in_specs=[
            pl.BlockSpec((2, 128, D), lambda i: (0, i, 0)),
            pl.BlockSpec((128, D), lambda i: (i, 0)),
            pl.BlockSpec((128, 1), lambda i: (i, 0)),
            pl.BlockSpec((1, D), lambda i: (0, 0)),
            pl.BlockSpec((D, D), lambda i: (0, 0)),
        ],
        out_specs=pl.BlockSpec((128, D), lambda i: (i, 0)),
        out_shape=jax.ShapeDtypeStruct((NPX, D), jnp.float32),
    )(sv, p, dinv, b, w)


def _final_body(s_ref, p_ref, dinv_ref, b_ref, batch_ref, o_ref,
                acc_ref, cnt_ref):
    i = pl.program_id(0)

    @pl.when(i == 0)
    def _():
        acc_ref[...] = jnp.zeros_like(acc_ref)
        cnt_ref[...] = jnp.zeros_like(cnt_ref)

    h = dinv_ref[...] * (s_ref[0] + s_ref[1] + p_ref[...]) + b_ref[...]
    gids = lax.broadcasted_iota(jnp.int32, (G, 1), 0)
    oh_t = (gids == batch_ref[0]).astype(jnp.float32)
    acc_ref[...] += jnp.dot(oh_t, h, preferred_element_type=jnp.float32,
                            precision=_HIGHEST)
    cnt_ref[...] += jnp.sum(oh_t, axis=1, keepdims=True)

    @pl.when(i == NPX // 128 - 1)
    def _():
        o_ref[...] = acc_ref[...] / jnp.maximum(cnt_ref[...], 1.0)


def _final_call(sv, p, dinv, b, batch3):
    return pl.pallas_call(
        _final_body,
        grid=(NPX // 128,),
        in_specs=[
            pl.BlockSpec((2, 128, D), lambda i: (0, i, 0)),
            pl.BlockSpec((128, D), lambda i: (i, 0)),
            pl.BlockSpec((128, 1), lambda i: (i, 0)),
            pl.BlockSpec((1, D), lambda i: (0, 0)),
            pl.BlockSpec((1, 1, 128), lambda i: (i, 0, 0)),
        ],
        out_specs=pl.BlockSpec((G, D), lambda i: (0, 0)),
        out_shape=jax.ShapeDtypeStruct((G, D), jnp.float32),
        scratch_shapes=[
            pltpu.VMEM((G, D), jnp.float32),
            pltpu.VMEM((G, 1), jnp.float32),
        ],
    )(sv, p, dinv, b, batch3)


def kernel(x, edge_index, batch, W1, b1, W2, b2, W3, b3):
    pad = jnp.full((EP - E,), N, dtype=jnp.int32)
    src3 = jnp.concatenate([edge_index[0], pad]).reshape(NW, NCHUNK, 128)
    dst3 = jnp.concatenate([edge_index[1], pad]).reshape(NW, NCHUNK, 128)
    srcc = src3.reshape(NW * NCHUNK, 128)
    dstc = dst3.reshape(NW * NCHUNK, 128)
    sd3 = jnp.stack([srcc, dstc], axis=1).reshape(2 * NW * NCHUNK, 128)
    xp = jnp.zeros((NPX, D), jnp.float32).at[:N].set(x)
    batch3 = jnp.concatenate(
        [batch, jnp.full((NPX - N,), G, dtype=jnp.int32)]
    ).reshape(NPX // 128, 1, 128)
    w3p = jnp.zeros((D, D), jnp.float32).at[:, :C].set(W3)
    b1r = b1.reshape(1, D)
    b2r = b2.reshape(1, D)
    b3r = jnp.zeros((1, D), jnp.float32).at[0, :C].set(b3)

    degs = _deg_kernel()(dst3)
    d0 = degs[:NPX].reshape(NPX, 1)
    d1 = degs[NPX:].reshape(NPX, 1)

    p1, dinv = _first_call(d0, d1, xp, W1)
    s1 = _scatter_kernel()(p1, sd3)
    p2 = _mid_call(s1, p1, dinv, b1r, W2)
    s2 = _scatter_kernel()(p2, sd3)
    p3 = _mid_call(s2, p2, dinv, b2r, w3p)
    s3 = _scatter_kernel()(p3, sd3)
    out = _final_call(s3, p3, dinv, b3r, batch3)
    return out[:, :C]


# speedup vs baseline: 1.2971x; 1.2971x over previous
import functools

import jax
import jax.numpy as jnp
from jax import lax
from jax.experimental import pallas as pl
from jax.experimental.pallas import tpu as pltpu
from jax.experimental.pallas import tpu_sc as plsc

N = 10000
E = 320000
D = 128
C = 40
G = 64

NPX = 10112
NCHUNK = 80
EP = 32 * NCHUNK * 128
NW = 32
RPT = NPX // 16
K0 = 120
K1 = 40

_HIGHEST = jax.lax.Precision.HIGHEST


@functools.lru_cache(maxsize=None)
def _sc_mesh():
    return plsc.VectorSubcoreMesh(core_axis_name="c", subcore_axis_name="s")


def _deg_body(dst_hbm, out_hbm, idx_d, ones_v, zeros_v, acc):
    c = lax.axis_index("c")
    s = lax.axis_index("s")
    wid = s * 2 + c
    pltpu.sync_copy(dst_hbm.at[wid], idx_d)
    for k in range(8):
        ones_v[pl.ds(k * 16, 16)] = jnp.ones((16,), jnp.float32)

    def _zfill(t, carry):
        zeros_v[pl.ds(t * 16, 16)] = jnp.zeros((16,), jnp.float32)
        return carry

    lax.fori_loop(0, 40, _zfill, 0)
    base = jnp.minimum(s * RPT, NPX - 640)
    pltpu.sync_copy(zeros_v, acc.at[pl.ds(base, 640)])
    plsc.subcore_barrier()

    def _scat(j, carry):
        pltpu.sync_copy(ones_v, acc.at[idx_d.at[j]], add=True)
        return carry

    lax.fori_loop(0, NCHUNK, _scat, 0)
    plsc.subcore_barrier()
    pltpu.sync_copy(acc.at[pl.ds(base, 640)], zeros_v)
    pltpu.sync_copy(zeros_v, out_hbm.at[pl.ds(c * NPX + base, 640)])


@functools.lru_cache(maxsize=None)
def _deg_kernel():
    return pl.kernel(
        _deg_body,
        out_type=jax.ShapeDtypeStruct((2 * NPX,), jnp.float32),
        mesh=_sc_mesh(),
        scratch_types=[
            pltpu.VMEM((NCHUNK, 128), jnp.int32),
            pltpu.VMEM((128,), jnp.float32),
            pltpu.VMEM((640,), jnp.float32),
            pltpu.VMEM_SHARED((NPX,), jnp.float32),
        ],
    )


def _scatter_body(p_hbm, sd_hbm, out_hbm, idx, rows, acc, gsem, ssem):
    c = lax.axis_index("c")
    s = lax.axis_index("s")
    wid = s * 2 + c

    def _zrow(r, carry):
        for k in range(8):
            rows[r, pl.ds(k * 16, 16)] = jnp.zeros((16,), jnp.float32)
        return carry

    lax.fori_loop(0, 128, _zrow, 0)

    def _zcopy(k, carry):
        pltpu.sync_copy(rows.at[pl.ds(0, 128)],
                        acc.at[pl.ds(s * RPT + k * 128, 128)])
        return carry

    lax.fori_loop(0, 4, _zcopy, 0)
    pltpu.sync_copy(rows.at[pl.ds(0, 120)],
                    acc.at[pl.ds(s * RPT + 512, 120)])
    plsc.subcore_barrier()

    rows_a = rows.at[pl.ds(0, 128)]
    rows_b = rows.at[pl.ds(128, 128)]
    start2 = pl.multiple_of(s * 320 + c * (2 * K0), 8)
    nph = jnp.where(c == 0, K0 // 8, K1 // 8)

    def _phase(ph, carry):
        off = pl.multiple_of(ph * 16, 16)
        pltpu.sync_copy(sd_hbm.at[pl.ds(start2 + off, 16)], idx)

        def _group(j, carry2):
            @pl.when((j > 0) | (ph > 0))
            def _():
                pltpu.make_async_copy(rows, acc.at[pl.ds(0, 256)],
                                      ssem).wait()

            r = j * 4
            pltpu.async_copy(p_hbm.at[idx.at[r]], rows_a, gsem)
            pltpu.async_copy(p_hbm.at[idx.at[r + 2]], rows_b, gsem)
            pltpu.make_async_copy(p_hbm.at[pl.ds(0, 256)], rows, gsem).wait()
            pltpu.async_copy(rows_a, acc.at[idx.at[r + 1]], ssem, add=True)
            pltpu.async_copy(rows_b, acc.at[idx.at[r + 3]], ssem, add=True)
            return carry2

        lax.fori_loop(0, 4, _group, 0)
        return carry

    lax.fori_loop(0, nph, _phase, 0)
    pltpu.make_async_copy(rows, acc.at[pl.ds(0, 256)], ssem).wait()
    plsc.subcore_barrier()

    def _wb(k, carry):
        pltpu.sync_copy(acc.at[pl.ds(s * RPT + k * 128, 128)],
                        out_hbm.at[c, pl.ds(s * RPT + k * 128, 128)])
        return carry

    lax.fori_loop(0, 4, _wb, 0)
    pltpu.sync_copy(acc.at[pl.ds(s * RPT + 512, 120)],
                    out_hbm.at[c, pl.ds(s * RPT + 512, 120)])


@functools.lru_cache(maxsize=None)
def _scatter_kernel():
    return pl.kernel(
        _scatter_body,
        out_type=jax.ShapeDtypeStruct((2, NPX, D), jnp.float32),
        mesh=_sc_mesh(),
        scratch_types=[
            pltpu.VMEM((16, 128), jnp.int32),
            pltpu.VMEM((256, D), jnp.float32),
            pltpu.VMEM_SHARED((NPX, D), jnp.float32),
            pltpu.SemaphoreType.DMA,
            pltpu.SemaphoreType.DMA,
        ],
    )


def _first_body(d0_ref, d1_ref, x_ref, w_ref, p_ref, dinv_ref):
    i = pl.program_id(0)
    deg = d0_ref[...] + d1_ref[...] + 1.0
    row = lax.broadcasted_iota(jnp.int32, (128, 1), 0) + i * 128
    dinv = jnp.where(row < N, lax.rsqrt(deg), 0.0)
    hw = jnp.dot(x_ref[...], w_ref[...],
                 preferred_element_type=jnp.float32, precision=_HIGHEST)
    p_ref[...] = dinv * hw
    dinv_ref[...] = dinv


def _first_call(d0, d1, x, w):
    return pl.pallas_call(
        _first_body,
        grid=(NPX // 128,),
        in_specs=[
            pl.BlockSpec((128, 1), lambda i: (i, 0)),
            pl.BlockSpec((128, 1), lambda i: (i, 0)),
            pl.BlockSpec((128, D), lambda i: (i, 0)),
            pl.BlockSpec((D, D), lambda i: (0, 0)),
        ],
        out_specs=[
            pl.BlockSpec((128, D), lambda i: (i, 0)),
            pl.BlockSpec((128, 1), lambda i: (i, 0)),
        ],
        out_shape=[
            jax.ShapeDtypeStruct((NPX, D), jnp.float32),
            jax.ShapeDtypeStruct((NPX, 1), jnp.float32),
        ],
    )(d0, d1, x, w)


def _mid_body(s_ref, p_ref, dinv_ref, b_ref, w_ref, o_ref):
    dinv = dinv_ref[...]
    h = jnp.maximum(dinv * (s_ref[0] + s_ref[1] + p_ref[...]) + b_ref[...],
                    0.0)
    o_ref[...] = dinv * jnp.dot(h, w_ref[...],
                                preferred_element_type=jnp.float32,
                                precision=_HIGHEST)


def _mid_call(sv, p, dinv, b, w):
    return pl.pallas_call(
        _mid_body,
        grid=(NPX // 128,),
        in_specs=[
            pl.BlockSpec((2, 128, D), lambda i: (0, i, 0)),
            pl.BlockSpec((128, D), lambda i: (i, 0)),
            pl.BlockSpec((128, 1), lambda i: (i, 0)),
            pl.BlockSpec((1, D), lambda i: (0, 0)),
            pl.BlockSpec((D, D), lambda i: (0, 0)),
        ],
        out_specs=pl.BlockSpec((128, D), lambda i: (i, 0)),
        out_shape=jax.ShapeDtypeStruct((NPX, D), jnp.float32),
    )(sv, p, dinv, b, w)


def _final_body(s_ref, p_ref, dinv_ref, b_ref, batch_ref, o_ref,
                acc_ref, cnt_ref):
    i = pl.program_id(0)

    @pl.when(i == 0)
    def _():
        acc_ref[...] = jnp.zeros_like(acc_ref)
        cnt_ref[...] = jnp.zeros_like(cnt_ref)

    h = dinv_ref[...] * (s_ref[0] + s_ref[1] + p_ref[...]) + b_ref[...]
    gids = lax.broadcasted_iota(jnp.int32, (G, 1), 0)
    oh_t = (gids == batch_ref[0]).astype(jnp.float32)
    acc_ref[...] += jnp.dot(oh_t, h, preferred_element_type=jnp.float32,
                            precision=_HIGHEST)
    cnt_ref[...] += jnp.sum(oh_t, axis=1, keepdims=True)

    @pl.when(i == NPX // 128 - 1)
    def _():
        o_ref[...] = acc_ref[...] / jnp.maximum(cnt_ref[...], 1.0)


def _final_call(sv, p, dinv, b, batch3):
    return pl.pallas_call(
        _final_body,
        grid=(NPX // 128,),
        in_specs=[
            pl.BlockSpec((2, 128, D), lambda i: (0, i, 0)),
            pl.BlockSpec((128, D), lambda i: (i, 0)),
            pl.BlockSpec((128, 1), lambda i: (i, 0)),
            pl.BlockSpec((1, D), lambda i: (0, 0)),
            pl.BlockSpec((1, 1, 128), lambda i: (i, 0, 0)),
        ],
        out_specs=pl.BlockSpec((G, D), lambda i: (0, 0)),
        out_shape=jax.ShapeDtypeStruct((G, D), jnp.float32),
        scratch_shapes=[
            pltpu.VMEM((G, D), jnp.float32),
            pltpu.VMEM((G, 1), jnp.float32),
        ],
    )(sv, p, dinv, b, batch3)


def kernel(x, edge_index, batch, W1, b1, W2, b2, W3, b3):
    pad = jnp.full((EP - E,), N, dtype=jnp.int32)
    src3 = jnp.concatenate([edge_index[0], pad]).reshape(NW, NCHUNK, 128)
    dst3 = jnp.concatenate([edge_index[1], pad]).reshape(NW, NCHUNK, 128)
    srcc = src3.reshape(NW * NCHUNK, 128)
    dstc = dst3.reshape(NW * NCHUNK, 128)
    sd3 = jnp.stack([srcc, dstc], axis=1).reshape(2 * NW * NCHUNK, 128)
    xp = jnp.zeros((NPX, D), jnp.float32).at[:N].set(x)
    batch3 = jnp.concatenate(
        [batch, jnp.full((NPX - N,), G, dtype=jnp.int32)]
    ).reshape(NPX // 128, 1, 128)
    w3p = jnp.zeros((D, D), jnp.float32).at[:, :C].set(W3)
    b1r = b1.reshape(1, D)
    b2r = b2.reshape(1, D)
    b3r = jnp.zeros((1, D), jnp.float32).at[0, :C].set(b3)

    degs = _deg_kernel()(dst3)
    d0 = degs[:NPX].reshape(NPX, 1)
    d1 = degs[NPX:].reshape(NPX, 1)

    p1, dinv = _first_call(d0, d1, xp, W1)
    s1 = _scatter_kernel()(p1, sd3)
    p2 = _mid_call(s1, p1, dinv, b1r, W2)
    s2 = _scatter_kernel()(p2, sd3)
    p3 = _mid_call(s2, p2, dinv, b2r, w3p)
    s3 = _scatter_kernel()(p3, sd3)
    out = _final_call(s3, p3, dinv, b3r, batch3)
    return out[:, :C]
